# baseline (speedup 1.0000x reference)
import jax
import jax.numpy as jnp
from jax import lax
from jax.experimental import pallas as pl
from jax.experimental.pallas import tpu as pltpu
from jax.experimental.pallas import tpu_sc as plsc

N_NODES = 10000
N_EDGES = 320000
D = 128

NUM_CORES = 2
NUM_SUBCORES = 16
NW = NUM_CORES * NUM_SUBCORES
EDGES_PER_WORKER = N_EDGES // NW
CHUNK = 40
NCHUNKS = EDGES_PER_WORKER // CHUNK
BLOCK = 10
NBLOCKS = NCHUNKS // BLOCK
DRAIN_ROWS = 40
DRAIN_CHUNKS = N_NODES // DRAIN_ROWS
NLANE = 16


def _node_proj_body(x_ref, w_ref, b_ref, h_ref, t_ref):
    p = jnp.dot(x_ref[...], w_ref[...], preferred_element_type=jnp.float32)
    p = p + b_ref[...]
    h_ref[...] = p[:, 0 * D:1 * D]
    t_ref[0] = p[:, 1 * D:2 * D]
    t_ref[1] = p[:, 2 * D:3 * D]
    t_ref[2] = p[:, 3 * D:4 * D]


def _node_proj(x, w, b):
    blk = 1000
    grid = N_NODES // blk
    return pl.pallas_call(
        _node_proj_body,
        grid=(grid,),
        in_specs=[
            pl.BlockSpec((blk, D), lambda i: (i, 0)),
            pl.BlockSpec((D, 4 * D), lambda i: (0, 0)),
            pl.BlockSpec((1, 4 * D), lambda i: (0, 0)),
        ],
        out_specs=[
            pl.BlockSpec((blk, D), lambda i: (i, 0)),
            pl.BlockSpec((3, blk, D), lambda i: (0, i, 0)),
        ],
        out_shape=[
            jax.ShapeDtypeStruct((N_NODES, D), jnp.float32),
            jax.ShapeDtypeStruct((3, N_NODES, D), jnp.float32),
        ],
    )(x, w, b.reshape(1, 4 * D))


def _edge_proj_body(ef_ref, we_ref, be_ref, o_ref):
    o_ref[...] = jnp.dot(ef_ref[...], we_ref[...],
                         preferred_element_type=jnp.float32) + be_ref[...]


def _edge_proj(ef, we, be):
    blk = 8000
    grid = N_EDGES // blk
    return pl.pallas_call(
        _edge_proj_body,
        grid=(grid,),
        in_specs=[
            pl.BlockSpec((blk, ef.shape[1]), lambda i: (i, 0)),
            pl.BlockSpec((ef.shape[1], D), lambda i: (0, 0)),
            pl.BlockSpec((1, D), lambda i: (0, 0)),
        ],
        out_specs=pl.BlockSpec((blk, D), lambda i: (i, 0)),
        out_shape=jax.ShapeDtypeStruct((N_EDGES, D), jnp.float32),
    )(ef, we, be.reshape(1, D))


def _sc_body(t_hbm, ep_hbm, c_hbm, r_hbm, out_hbm,
             cidxB, ridxB, gb0, gb1, mb, eb,
             acc, sem0, sem1, sem2):
    cid = lax.axis_index("c")
    sid = lax.axis_index("s")
    wid = cid * NUM_SUBCORES + sid

    @pl.loop(0, DRAIN_ROWS)
    def _(i):
        for j in range(D // NLANE):
            mb[i, pl.ds(j * NLANE, NLANE)] = jnp.zeros((NLANE,), jnp.float32)

    for t in range(-(-DRAIN_CHUNKS // NUM_SUBCORES)):
        c = sid + t * NUM_SUBCORES

        @pl.when(c < DRAIN_CHUNKS)
        def _():
            off = pl.multiple_of(c * DRAIN_ROWS, 8)
            pltpu.sync_copy(mb, acc.at[pl.ds(off, DRAIN_ROWS)])

    plsc.subcore_barrier()

    gbufs = ((gb0, sem0), (gb1, sem1))

    @pl.loop(0, NBLOCKS)
    def _(blk):
        pltpu.sync_copy(c_hbm.at[wid * NBLOCKS + blk], cidxB)
        pltpu.sync_copy(r_hbm.at[wid * NBLOCKS + blk], ridxB)
        pltpu.async_copy(t_hbm.at[cidxB.at[0]], gb0, sem0)

        @pl.loop(0, BLOCK, step=2)
        def _(i):
            for b in range(2):
                r = i + b

                @pl.when(r + 1 < BLOCK)
                def _():
                    gb, sem = gbufs[1 - b]
                    pltpu.async_copy(t_hbm.at[cidxB.at[r + 1]], gb, sem)

                gb, sem = gbufs[b]
                base = pl.multiple_of(
                    (wid * EDGES_PER_WORKER + (blk * BLOCK + r) * CHUNK), 8)
                de = pltpu.async_copy(ep_hbm.at[pl.ds(base, CHUNK)], eb, sem2)
                pltpu.make_async_copy(t_hbm.at[cidxB.at[r]], gb, sem).wait()
                de.wait()

                @pl.loop(0, CHUNK)
                def _(e):
                    for j in range(D // NLANE):
                        sl = pl.ds(j * NLANE, NLANE)
                        x = gb[e, sl] + gb[CHUNK + e, sl] + eb[e, sl]
                        eta = 1.0 / (1.0 + jnp.exp(-x))
                        mb[e, sl] = eta * gb[2 * CHUNK + e, sl]

                pltpu.sync_copy(mb, acc.at[ridxB.at[r]], add=True)

    plsc.subcore_barrier()

    for t in range(-(-DRAIN_CHUNKS // NUM_SUBCORES)):
        c = sid + t * NUM_SUBCORES

        @pl.when(c < DRAIN_CHUNKS)
        def _():
            off = pl.multiple_of(c * DRAIN_ROWS, 8)
            rows = pl.ds(off, DRAIN_ROWS)
            pltpu.sync_copy(acc.at[rows], mb)
            pltpu.sync_copy(mb, out_hbm.at[cid, rows])


def _sc_gather_scatter(t3, ep, senders, receivers):
    t = t3.reshape(3 * N_NODES, D)
    s2 = senders.reshape(N_EDGES // CHUNK, CHUNK)
    r2 = receivers.reshape(N_EDGES // CHUNK, CHUNK)
    cidx = jnp.stack([r2, s2 + N_NODES, s2 + 2 * N_NODES], axis=1)
    cidx = cidx.reshape(NW * NBLOCKS, BLOCK, 3 * CHUNK)
    r3 = r2.reshape(NW * NBLOCKS, BLOCK, CHUNK)
    mesh = plsc.VectorSubcoreMesh(core_axis_name="c", subcore_axis_name="s")
    kern = pl.kernel(
        _sc_body,
        mesh=mesh,
        out_type=jax.ShapeDtypeStruct((NUM_CORES, N_NODES, D), jnp.float32),
        scratch_types=[
            pltpu.VMEM((BLOCK, 3 * CHUNK), jnp.int32),
            pltpu.VMEM((BLOCK, CHUNK), jnp.int32),
            pltpu.VMEM((3 * CHUNK, D), jnp.float32),
            pltpu.VMEM((3 * CHUNK, D), jnp.float32),
            pltpu.VMEM((CHUNK, D), jnp.float32),
            pltpu.VMEM((CHUNK, D), jnp.float32),
            pltpu.VMEM_SHARED((N_NODES, D), jnp.float32),
            pltpu.SemaphoreType.DMA,
            pltpu.SemaphoreType.DMA,
            pltpu.SemaphoreType.DMA,
        ],
    )
    return kern(t, ep, cidx, r3)


def _combine_body(h_ref, p_ref, o_ref):
    o_ref[...] = h_ref[...] + p_ref[0] + p_ref[1]


def _combine(h, partials):
    blk = 1000
    grid = N_NODES // blk
    return pl.pallas_call(
        _combine_body,
        grid=(grid,),
        in_specs=[
            pl.BlockSpec((blk, D), lambda i: (i, 0)),
            pl.BlockSpec((NUM_CORES, blk, D), lambda i: (0, i, 0)),
        ],
        out_specs=pl.BlockSpec((blk, D), lambda i: (i, 0)),
        out_shape=jax.ShapeDtypeStruct((N_NODES, D), jnp.float32),
    )(h, partials)


@jax.jit
def kernel(node_features, senders, receivers, edge_features,
           W_kernel, W_bias, We_kernel, We_bias):
    h, t3 = _node_proj(node_features, W_kernel, W_bias)
    ep = _edge_proj(edge_features, We_kernel, We_bias)
    partials = _sc_gather_scatter(t3, ep, senders, receivers)
    return _combine(h, partials)


# speedup vs baseline: 1.2082x; 1.2082x over previous
import jax
import jax.numpy as jnp
from jax import lax
from jax.experimental import pallas as pl
from jax.experimental.pallas import tpu as pltpu
from jax.experimental.pallas import tpu_sc as plsc

N_NODES = 10000
N_EDGES = 320000
D = 128

NUM_CORES = 2
NUM_SUBCORES = 16
NW = NUM_CORES * NUM_SUBCORES
EDGES_PER_WORKER = N_EDGES // NW
CHUNK = 40
NCHUNKS = EDGES_PER_WORKER // CHUNK
BLOCK = 10
NBLOCKS = NCHUNKS // BLOCK
DRAIN_ROWS = 40
DRAIN_CHUNKS = N_NODES // DRAIN_ROWS
NLANE = 16


def _node_proj_body(x_ref, w_ref, b_ref, h_ref, t_ref):
    p = jnp.dot(x_ref[...], w_ref[...], preferred_element_type=jnp.float32)
    p = p + b_ref[...]
    h_ref[...] = p[:, 0 * D:1 * D]
    t_ref[0] = p[:, 1 * D:2 * D]
    t_ref[1] = p[:, 2 * D:3 * D]
    t_ref[2] = p[:, 3 * D:4 * D]


def _node_proj(x, w, b):
    blk = 1000
    grid = N_NODES // blk
    return pl.pallas_call(
        _node_proj_body,
        grid=(grid,),
        in_specs=[
            pl.BlockSpec((blk, D), lambda i: (i, 0)),
            pl.BlockSpec((D, 4 * D), lambda i: (0, 0)),
            pl.BlockSpec((1, 4 * D), lambda i: (0, 0)),
        ],
        out_specs=[
            pl.BlockSpec((blk, D), lambda i: (i, 0)),
            pl.BlockSpec((3, blk, D), lambda i: (0, i, 0)),
        ],
        out_shape=[
            jax.ShapeDtypeStruct((N_NODES, D), jnp.float32),
            jax.ShapeDtypeStruct((3, N_NODES, D), jnp.float32),
        ],
    )(x, w, b.reshape(1, 4 * D))


def _edge_proj_body(ef_ref, we_ref, be_ref, o_ref):
    o_ref[...] = jnp.dot(ef_ref[...], we_ref[...],
                         preferred_element_type=jnp.float32) + be_ref[...]


def _edge_proj(ef, we, be):
    blk = 8000
    grid = N_EDGES // blk
    return pl.pallas_call(
        _edge_proj_body,
        grid=(grid,),
        in_specs=[
            pl.BlockSpec((blk, ef.shape[1]), lambda i: (i, 0)),
            pl.BlockSpec((ef.shape[1], D), lambda i: (0, 0)),
            pl.BlockSpec((1, D), lambda i: (0, 0)),
        ],
        out_specs=pl.BlockSpec((blk, D), lambda i: (i, 0)),
        out_shape=jax.ShapeDtypeStruct((N_EDGES, D), jnp.float32),
    )(ef, we, be.reshape(1, D))


def _sc_body(t_hbm, ep_hbm, c_hbm, r_hbm, out_hbm,
             cidxB, ridxB, gb0, gb1, eb0, eb1,
             acc, sem0, sem1):
    cid = lax.axis_index("c")
    sid = lax.axis_index("s")
    wid = cid * NUM_SUBCORES + sid

    @pl.loop(0, DRAIN_ROWS)
    def _(i):
        for j in range(D // NLANE):
            eb0[i, pl.ds(j * NLANE, NLANE)] = jnp.zeros((NLANE,), jnp.float32)

    for t in range(-(-DRAIN_CHUNKS // NUM_SUBCORES)):
        c = sid + t * NUM_SUBCORES

        @pl.when(c < DRAIN_CHUNKS)
        def _():
            off = pl.multiple_of(c * DRAIN_ROWS, 8)
            pltpu.sync_copy(eb0, acc.at[pl.ds(off, DRAIN_ROWS)])

    plsc.subcore_barrier()

    gbufs = ((gb0, eb0, sem0), (gb1, eb1, sem1))

    def fire(blk, row, b):
        gb, eb, sem = gbufs[b]
        base = pl.multiple_of(
            (wid * EDGES_PER_WORKER + (blk * BLOCK + row) * CHUNK), 8)
        pltpu.async_copy(t_hbm.at[cidxB.at[row]], gb, sem)
        pltpu.async_copy(ep_hbm.at[pl.ds(base, CHUNK)], eb, sem)

    @pl.loop(0, NBLOCKS)
    def _(blk):
        pltpu.sync_copy(c_hbm.at[wid * NBLOCKS + blk], cidxB)
        pltpu.sync_copy(r_hbm.at[wid * NBLOCKS + blk], ridxB)
        fire(blk, 0, 0)

        @pl.loop(0, BLOCK, step=2)
        def _(i):
            for b in range(2):
                r = i + b

                @pl.when(r + 1 < BLOCK)
                def _():
                    fire(blk, r + 1, 1 - b)

                gb, eb, sem = gbufs[b]
                base = pl.multiple_of(
                    (wid * EDGES_PER_WORKER + (blk * BLOCK + r) * CHUNK), 8)
                pltpu.make_async_copy(t_hbm.at[cidxB.at[r]], gb, sem).wait()
                pltpu.make_async_copy(
                    ep_hbm.at[pl.ds(base, CHUNK)], eb, sem).wait()

                @pl.loop(0, CHUNK)
                def _(e):
                    for j in range(D // NLANE):
                        sl = pl.ds(j * NLANE, NLANE)
                        x = gb[e, sl] + gb[CHUNK + e, sl] + eb[e, sl]
                        eta = 1.0 / (1.0 + jnp.exp(-x))
                        gb[e, sl] = eta * gb[2 * CHUNK + e, sl]

                pltpu.sync_copy(gb.at[pl.ds(0, CHUNK)],
                                acc.at[ridxB.at[r]], add=True)

    plsc.subcore_barrier()

    for t in range(-(-DRAIN_CHUNKS // NUM_SUBCORES)):
        c = sid + t * NUM_SUBCORES

        @pl.when(c < DRAIN_CHUNKS)
        def _():
            off = pl.multiple_of(c * DRAIN_ROWS, 8)
            rows = pl.ds(off, DRAIN_ROWS)
            pltpu.sync_copy(acc.at[rows], eb0)
            pltpu.sync_copy(eb0, out_hbm.at[cid, rows])


def _sc_gather_scatter(t3, ep, senders, receivers):
    t = t3.reshape(3 * N_NODES, D)
    s2 = senders.reshape(N_EDGES // CHUNK, CHUNK)
    r2 = receivers.reshape(N_EDGES // CHUNK, CHUNK)
    cidx = jnp.stack([r2, s2 + N_NODES, s2 + 2 * N_NODES], axis=1)
    cidx = cidx.reshape(NW * NBLOCKS, BLOCK, 3 * CHUNK)
    r3 = r2.reshape(NW * NBLOCKS, BLOCK, CHUNK)
    mesh = plsc.VectorSubcoreMesh(core_axis_name="c", subcore_axis_name="s")
    kern = pl.kernel(
        _sc_body,
        mesh=mesh,
        out_type=jax.ShapeDtypeStruct((NUM_CORES, N_NODES, D), jnp.float32),
        scratch_types=[
            pltpu.VMEM((BLOCK, 3 * CHUNK), jnp.int32),
            pltpu.VMEM((BLOCK, CHUNK), jnp.int32),
            pltpu.VMEM((3 * CHUNK, D), jnp.float32),
            pltpu.VMEM((3 * CHUNK, D), jnp.float32),
            pltpu.VMEM((CHUNK, D), jnp.float32),
            pltpu.VMEM((CHUNK, D), jnp.float32),
            pltpu.VMEM_SHARED((N_NODES, D), jnp.float32),
            pltpu.SemaphoreType.DMA,
            pltpu.SemaphoreType.DMA,
        ],
    )
    return kern(t, ep, cidx, r3)


def _combine_body(h_ref, p_ref, o_ref):
    o_ref[...] = h_ref[...] + p_ref[0] + p_ref[1]


def _combine(h, partials):
    blk = 1000
    grid = N_NODES // blk
    return pl.pallas_call(
        _combine_body,
        grid=(grid,),
        in_specs=[
            pl.BlockSpec((blk, D), lambda i: (i, 0)),
            pl.BlockSpec((NUM_CORES, blk, D), lambda i: (0, i, 0)),
        ],
        out_specs=pl.BlockSpec((blk, D), lambda i: (i, 0)),
        out_shape=jax.ShapeDtypeStruct((N_NODES, D), jnp.float32),
    )(h, partials)


@jax.jit
def kernel(node_features, senders, receivers, edge_features,
           W_kernel, W_bias, We_kernel, We_bias):
    h, t3 = _node_proj(node_features, W_kernel, W_bias)
    ep = _edge_proj(edge_features, We_kernel, We_bias)
    partials = _sc_gather_scatter(t3, ep, senders, receivers)
    return _combine(h, partials)


# speedup vs baseline: 1.2175x; 1.0077x over previous
import jax
import jax.numpy as jnp
from jax import lax
from jax.experimental import pallas as pl
from jax.experimental.pallas import tpu as pltpu
from jax.experimental.pallas import tpu_sc as plsc

N_NODES = 10000
N_EDGES = 320000
D = 128

NUM_CORES = 2
NUM_SUBCORES = 16
NW = NUM_CORES * NUM_SUBCORES
EDGES_PER_WORKER = N_EDGES // NW
CHUNK = 40
NCHUNKS = EDGES_PER_WORKER // CHUNK
BLOCK = 10
NBLOCKS = NCHUNKS // BLOCK
DRAIN_ROWS = 40
DRAIN_CHUNKS = N_NODES // DRAIN_ROWS
NLANE = 16


def _node_proj_body(x_ref, w_ref, b_ref, h_ref, t_ref):
    p = jnp.dot(x_ref[...], w_ref[...], preferred_element_type=jnp.float32)
    p = p + b_ref[...]
    h_ref[...] = p[:, 0 * D:1 * D]
    t_ref[0] = p[:, 1 * D:2 * D]
    t_ref[1] = p[:, 2 * D:3 * D]
    t_ref[2] = p[:, 3 * D:4 * D]


def _node_proj(x, w, b):
    blk = 1000
    grid = N_NODES // blk
    return pl.pallas_call(
        _node_proj_body,
        grid=(grid,),
        in_specs=[
            pl.BlockSpec((blk, D), lambda i: (i, 0)),
            pl.BlockSpec((D, 4 * D), lambda i: (0, 0)),
            pl.BlockSpec((1, 4 * D), lambda i: (0, 0)),
        ],
        out_specs=[
            pl.BlockSpec((blk, D), lambda i: (i, 0)),
            pl.BlockSpec((3, blk, D), lambda i: (0, i, 0)),
        ],
        out_shape=[
            jax.ShapeDtypeStruct((N_NODES, D), jnp.float32),
            jax.ShapeDtypeStruct((3, N_NODES, D), jnp.float32),
        ],
    )(x, w, b.reshape(1, 4 * D))


def _edge_proj_body(ef_ref, we_ref, be_ref, o_ref):
    o_ref[...] = jnp.dot(ef_ref[...], we_ref[...],
                         preferred_element_type=jnp.float32) + be_ref[...]


def _edge_proj(ef, we, be):
    blk = 8000
    grid = N_EDGES // blk
    return pl.pallas_call(
        _edge_proj_body,
        grid=(grid,),
        in_specs=[
            pl.BlockSpec((blk, ef.shape[1]), lambda i: (i, 0)),
            pl.BlockSpec((ef.shape[1], D), lambda i: (0, 0)),
            pl.BlockSpec((1, D), lambda i: (0, 0)),
        ],
        out_specs=pl.BlockSpec((blk, D), lambda i: (i, 0)),
        out_shape=jax.ShapeDtypeStruct((N_EDGES, D), jnp.float32),
    )(ef, we, be.reshape(1, D))


def _sc_body(t_hbm, ep_hbm, c_hbm, r_hbm, out_hbm,
             cidxB, ridxB, gb0, gb1, eb0, eb1,
             acc, sem0, sem1, sem3):
    cid = lax.axis_index("c")
    sid = lax.axis_index("s")
    wid = cid * NUM_SUBCORES + sid

    @pl.loop(0, DRAIN_ROWS)
    def _(i):
        for j in range(D // NLANE):
            eb0[i, pl.ds(j * NLANE, NLANE)] = jnp.zeros((NLANE,), jnp.float32)

    for t in range(-(-DRAIN_CHUNKS // NUM_SUBCORES)):
        c = sid + t * NUM_SUBCORES

        @pl.when(c < DRAIN_CHUNKS)
        def _():
            off = pl.multiple_of(c * DRAIN_ROWS, 8)
            pltpu.sync_copy(eb0, acc.at[pl.ds(off, DRAIN_ROWS)])

    plsc.subcore_barrier()

    gbufs = ((gb0, eb0, sem0), (gb1, eb1, sem1))

    def fire(blk, row, b):
        gb, eb, sem = gbufs[b]
        base = pl.multiple_of(
            (wid * EDGES_PER_WORKER + (blk * BLOCK + row) * CHUNK), 8)
        pltpu.async_copy(t_hbm.at[cidxB.at[row]], gb, sem)
        pltpu.async_copy(ep_hbm.at[pl.ds(base, CHUNK)], eb, sem)

    @pl.loop(0, NBLOCKS)
    def _(blk):
        pltpu.sync_copy(c_hbm.at[wid * NBLOCKS + blk], cidxB)
        pltpu.sync_copy(r_hbm.at[wid * NBLOCKS + blk], ridxB)

        @pl.when(blk > 0)
        def _():
            pltpu.make_async_copy(gb0.at[pl.ds(0, CHUNK)],
                                  acc.at[ridxB.at[0]], sem3).wait()

        fire(blk, 0, 0)

        @pl.loop(0, BLOCK, step=2)
        def _(i):
            for b in range(2):
                r = i + b

                @pl.when(r + 1 < BLOCK)
                def _():
                    @pl.when((blk > 0) | (r > 0))
                    def _():
                        pltpu.make_async_copy(gb0.at[pl.ds(0, CHUNK)],
                                              acc.at[ridxB.at[0]],
                                              sem3).wait()

                    fire(blk, r + 1, 1 - b)

                gb, eb, sem = gbufs[b]
                base = pl.multiple_of(
                    (wid * EDGES_PER_WORKER + (blk * BLOCK + r) * CHUNK), 8)
                pltpu.make_async_copy(t_hbm.at[cidxB.at[r]], gb, sem).wait()
                pltpu.make_async_copy(
                    ep_hbm.at[pl.ds(base, CHUNK)], eb, sem).wait()

                @pl.loop(0, CHUNK)
                def _(e):
                    for j in range(D // NLANE):
                        sl = pl.ds(j * NLANE, NLANE)
                        x = gb[e, sl] + gb[CHUNK + e, sl] + eb[e, sl]
                        eta = 1.0 / (1.0 + jnp.exp(-x))
                        gb[e, sl] = eta * gb[2 * CHUNK + e, sl]

                pltpu.async_copy(gb.at[pl.ds(0, CHUNK)],
                                 acc.at[ridxB.at[r]], sem3, add=True)

    for _ in range(2):
        pltpu.make_async_copy(gb0.at[pl.ds(0, CHUNK)],
                              acc.at[ridxB.at[0]], sem3).wait()

    plsc.subcore_barrier()

    for t in range(-(-DRAIN_CHUNKS // NUM_SUBCORES)):
        c = sid + t * NUM_SUBCORES

        @pl.when(c < DRAIN_CHUNKS)
        def _():
            off = pl.multiple_of(c * DRAIN_ROWS, 8)
            rows = pl.ds(off, DRAIN_ROWS)
            pltpu.sync_copy(acc.at[rows], eb0)
            pltpu.sync_copy(eb0, out_hbm.at[cid, rows])


def _sc_gather_scatter(t3, ep, senders, receivers):
    t = t3.reshape(3 * N_NODES, D)
    s2 = senders.reshape(N_EDGES // CHUNK, CHUNK)
    r2 = receivers.reshape(N_EDGES // CHUNK, CHUNK)
    cidx = jnp.stack([r2, s2 + N_NODES, s2 + 2 * N_NODES], axis=1)
    cidx = cidx.reshape(NW * NBLOCKS, BLOCK, 3 * CHUNK)
    r3 = r2.reshape(NW * NBLOCKS, BLOCK, CHUNK)
    mesh = plsc.VectorSubcoreMesh(core_axis_name="c", subcore_axis_name="s")
    kern = pl.kernel(
        _sc_body,
        mesh=mesh,
        out_type=jax.ShapeDtypeStruct((NUM_CORES, N_NODES, D), jnp.float32),
        scratch_types=[
            pltpu.VMEM((BLOCK, 3 * CHUNK), jnp.int32),
            pltpu.VMEM((BLOCK, CHUNK), jnp.int32),
            pltpu.VMEM((3 * CHUNK, D), jnp.float32),
            pltpu.VMEM((3 * CHUNK, D), jnp.float32),
            pltpu.VMEM((CHUNK, D), jnp.float32),
            pltpu.VMEM((CHUNK, D), jnp.float32),
            pltpu.VMEM_SHARED((N_NODES, D), jnp.float32),
            pltpu.SemaphoreType.DMA,
            pltpu.SemaphoreType.DMA,
            pltpu.SemaphoreType.DMA,
        ],
    )
    return kern(t, ep, cidx, r3)


def _combine_body(h_ref, p_ref, o_ref):
    o_ref[...] = h_ref[...] + p_ref[0] + p_ref[1]


def _combine(h, partials):
    blk = 1000
    grid = N_NODES // blk
    return pl.pallas_call(
        _combine_body,
        grid=(grid,),
        in_specs=[
            pl.BlockSpec((blk, D), lambda i: (i, 0)),
            pl.BlockSpec((NUM_CORES, blk, D), lambda i: (0, i, 0)),
        ],
        out_specs=pl.BlockSpec((blk, D), lambda i: (i, 0)),
        out_shape=jax.ShapeDtypeStruct((N_NODES, D), jnp.float32),
    )(h, partials)


@jax.jit
def kernel(node_features, senders, receivers, edge_features,
           W_kernel, W_bias, We_kernel, We_bias):
    h, t3 = _node_proj(node_features, W_kernel, W_bias)
    ep = _edge_proj(edge_features, We_kernel, We_bias)
    partials = _sc_gather_scatter(t3, ep, senders, receivers)
    return _combine(h, partials)
